# Initial kernel scaffold; baseline (speedup 1.0000x reference)
import jax
import jax.numpy as jnp
from jax.experimental import pallas as pl


def kernel(pred_s, pred_t, delta_e, vertex_idx, edge_idx):
    raise NotImplementedError("write your pallas kernel here")


# speedup vs baseline: 8.9443x; 8.9443x over previous
import functools

import jax
import jax.numpy as jnp
from jax import lax
from jax.experimental import pallas as pl
from jax.experimental.pallas import tpu as pltpu
from jax.experimental.pallas import tpu_sc as plsc

N_NODES = 50000
N_EDGES = 25000
NNZ = 800000
C = 64

E_PAD = 25088
ROWS_PER_SUB = E_PAD // 16
NNZ_PAD = 802816
CHUNK = 128
N_CHUNK_ROWS = NNZ_PAD // CHUNK
CHUNKS_PER_SUB = N_CHUNK_ROWS // 16
IDX_BLOCK = 28


def _softmax_body(x_ref, y_ref, ox_ref, oy_ref):
    for r, o in ((x_ref, ox_ref), (y_ref, oy_ref)):
        x = r[...]
        e = jnp.exp(x - jnp.max(x, axis=1, keepdims=True))
        o[...] = e / jnp.sum(e, axis=1, keepdims=True)


def _softmax(pred_s, pred_t):
    blk = 2000
    grid = N_NODES // blk
    spec = pl.BlockSpec((blk, C), lambda i: (i, 0))
    return pl.pallas_call(
        _softmax_body,
        grid=(grid,),
        in_specs=[spec, spec],
        out_specs=[spec, spec],
        out_shape=[
            jax.ShapeDtypeStruct((N_NODES, C), jnp.float32),
            jax.ShapeDtypeStruct((N_NODES, C), jnp.float32),
        ],
    )(pred_s, pred_t)


def _agg_body(ts, tt, vidx, eidx, ss, st, cnt,
              s_sp, cnt_sp, vbuf, ebuf, rows, zb, zc, ones, sem):
    cid = lax.axis_index("c")
    sid = lax.axis_index("s")
    base = sid * ROWS_PER_SUB

    def _fill_zb(i, _):
        for c4 in range(4):
            zb[i, pl.ds(c4 * 16, 16)] = jnp.zeros((16,), jnp.float32)
        return 0
    lax.fori_loop(0, 98, _fill_zb, 0)

    def _fill_zc(i, _):
        zc[pl.ds(i * 16, 16)] = jnp.zeros((16,), jnp.float32)
        return 0
    lax.fori_loop(0, ROWS_PER_SUB // 16, _fill_zc, 0)

    def _fill_ones(i, _):
        ones[pl.ds(i * 16, 16)] = jnp.full((16,), 1.0, jnp.float32)
        return 0
    lax.fori_loop(0, CHUNK // 16, _fill_ones, 0)

    for b in range(16):
        pltpu.sync_copy(zb, s_sp.at[pl.ds(base + b * 98, 98)])

    @pl.when(cid == 0)
    def _():
        pltpu.sync_copy(zc, cnt_sp.at[pl.ds(base, ROWS_PER_SUB)])

    plsc.subcore_barrier()

    row0 = sid * CHUNKS_PER_SUB
    for blk in range(CHUNKS_PER_SUB // IDX_BLOCK):
        pltpu.sync_copy(vidx.at[pl.ds(row0 + blk * IDX_BLOCK, IDX_BLOCK)], vbuf)
        pltpu.sync_copy(eidx.at[pl.ds(row0 + blk * IDX_BLOCK, IDX_BLOCK)], ebuf)

        def _chunk(j, _):
            vrow = vbuf.at[j]
            erow = ebuf.at[j]

            @pl.when(cid == 0)
            def _():
                pltpu.async_copy(ts.at[vrow], rows, sem).wait()
                pltpu.sync_copy(rows, s_sp.at[erow], add=True)
                pltpu.sync_copy(ones, cnt_sp.at[erow], add=True)

            @pl.when(cid == 1)
            def _():
                pltpu.async_copy(tt.at[vrow], rows, sem).wait()
                pltpu.sync_copy(rows, s_sp.at[erow], add=True)

            return 0
        lax.fori_loop(0, IDX_BLOCK, _chunk, 0)

    plsc.subcore_barrier()

    @pl.when(cid == 0)
    def _():
        pltpu.sync_copy(s_sp.at[pl.ds(base, ROWS_PER_SUB)],
                        ss.at[pl.ds(base, ROWS_PER_SUB)])
        pltpu.sync_copy(cnt_sp.at[pl.ds(base, ROWS_PER_SUB)],
                        cnt.at[pl.ds(base, ROWS_PER_SUB)])

    @pl.when(cid == 1)
    def _():
        pltpu.sync_copy(s_sp.at[pl.ds(base, ROWS_PER_SUB)],
                        st.at[pl.ds(base, ROWS_PER_SUB)])


def _aggregate(ps, pt, vidx2d, eidx2d):
    mesh = plsc.VectorSubcoreMesh(core_axis_name="c", subcore_axis_name="s")
    k = pl.kernel(
        _agg_body,
        out_type=(
            jax.ShapeDtypeStruct((E_PAD, C), jnp.float32),
            jax.ShapeDtypeStruct((E_PAD, C), jnp.float32),
            jax.ShapeDtypeStruct((E_PAD,), jnp.float32),
        ),
        mesh=mesh,
        compiler_params=pltpu.CompilerParams(use_tc_tiling_on_sc=False),
        scratch_types=[
            pltpu.VMEM_SHARED((E_PAD, C), jnp.float32),
            pltpu.VMEM_SHARED((E_PAD,), jnp.float32),
            pltpu.VMEM((IDX_BLOCK, CHUNK), jnp.int32),
            pltpu.VMEM((IDX_BLOCK, CHUNK), jnp.int32),
            pltpu.VMEM((CHUNK, C), jnp.float32),
            pltpu.VMEM((98, C), jnp.float32),
            pltpu.VMEM((ROWS_PER_SUB,), jnp.float32),
            pltpu.VMEM((CHUNK,), jnp.float32),
            pltpu.SemaphoreType.DMA,
        ],
    )
    return k(ps, pt, vidx2d, eidx2d)


E_BLK = E_PAD // 16


def _loss_body(ss_ref, st_ref, cnt_ref, de_ref, u_ref, out_ref, acc):
    i = pl.program_id(0)

    @pl.when(i == 0)
    def _():
        acc[0] = 0.0
        acc[1] = 0.0

    c = jnp.maximum(cnt_ref[...], 1.0)
    ms = ss_ref[...] / c
    mt = st_ref[...] / c
    per = jnp.sum(jnp.exp(mt) * (mt - jnp.log(ms)), axis=1, keepdims=True)
    mask = u_ref[...] < de_ref[...]
    acc[0] += jnp.sum(jnp.where(mask, per, 0.0))
    acc[1] += jnp.sum(jnp.where(mask, 1.0, 0.0))

    @pl.when(i == pl.num_programs(0) - 1)
    def _():
        out_ref[0, 0] = acc[0] / jnp.maximum(acc[1], 1.0)


def _loss(ss, st, cnt, de, u):
    mat = pl.BlockSpec((E_BLK, C), lambda i: (i, 0))
    vec = pl.BlockSpec((E_BLK, 1), lambda i: (i, 0))
    return pl.pallas_call(
        _loss_body,
        grid=(E_PAD // E_BLK,),
        in_specs=[mat, mat, vec, vec, vec],
        out_specs=pl.BlockSpec((1, 1), lambda i: (0, 0), memory_space=pltpu.SMEM),
        out_shape=jax.ShapeDtypeStruct((1, 1), jnp.float32),
        scratch_shapes=[pltpu.SMEM((2,), jnp.float32)],
    )(ss, st, cnt, de, u)


def kernel(pred_s, pred_t, delta_e, vertex_idx, edge_idx):
    ps, pt = _softmax(pred_s, pred_t)

    pad = NNZ_PAD - NNZ
    vidx2d = jnp.concatenate(
        [vertex_idx, jnp.zeros((pad,), jnp.int32)]).reshape(N_CHUNK_ROWS, CHUNK)
    eidx2d = jnp.concatenate(
        [edge_idx, jnp.full((pad,), N_EDGES, jnp.int32)]).reshape(N_CHUNK_ROWS, CHUNK)

    ss, st, cnt = _aggregate(ps, pt, vidx2d, eidx2d)

    u = jax.random.uniform(jax.random.key(42), (N_EDGES,), jnp.float32)
    epad = E_PAD - N_EDGES
    u2 = jnp.concatenate([u, jnp.full((epad,), 2.0, jnp.float32)]).reshape(E_PAD, 1)
    de2 = jnp.concatenate([delta_e, jnp.zeros((epad,), jnp.float32)]).reshape(E_PAD, 1)

    loss = _loss(ss, st, cnt.reshape(E_PAD, 1), de2, u2)
    return loss.reshape(())
